# baseline (speedup 1.0000x reference)
import functools

import jax
import jax.numpy as jnp
from jax import lax
from jax.experimental import pallas as pl
from jax.experimental.pallas import tpu as pltpu
from jax.experimental.pallas import tpu_sc as plsc

NBINS = 10001
HB = 10240
HTOT = 2 * HB
DEAD = HB - 1
NC = 1
NS = 16
NW = NC * NS
EPW = 6400
NPAD = NW * EPW
NCHUNK = EPW // 128
NVEC = EPW // 16


def _hist_kernel(preds_hbm, targs_hbm, out_hbm, pred_v, targ_v, idx_v,
                 ones_v, zero_v, shist, sem):
    c = lax.axis_index("c")
    s = lax.axis_index("s")
    wid = s * NC + c
    base = wid * EPW

    pltpu.sync_copy(preds_hbm.at[pl.ds(base, EPW)], pred_v)
    pltpu.sync_copy(targs_hbm.at[pl.ds(base, EPW)], targ_v)

    ones16 = jnp.full((16,), 1.0, dtype=jnp.float32)
    zero16 = jnp.zeros((16,), dtype=jnp.float32)
    for k in range(128 // 16):
        ones_v[pl.ds(k * 16, 16)] = ones16

    def _zinit(i, _):
        zero_v[pl.ds(i * 16, 16)] = zero16
        return 0
    lax.fori_loop(0, (HTOT // NS) // 16, _zinit, 0)

    stripe = HTOT // NS
    pltpu.sync_copy(zero_v, shist.at[pl.ds(s * stripe, stripe)])

    def _step(i, _):
        p = pred_v[pl.ds(i * 16, 16)]
        t = targ_v[pl.ds(i * 16, 16)]
        sg = 1.0 / (1.0 + jnp.exp(-p))
        b = (10000.0 * sg).astype(jnp.int32)
        idx = jnp.where(t >= 0.5, b + HB, b)
        idx = jnp.where(t < -0.5, DEAD, idx)
        row = i // 8
        col = (i % 8) * 16
        idx_v[row, pl.ds(col, 16)] = idx
        return 0
    lax.fori_loop(0, NVEC, _step, 0)

    plsc.subcore_barrier()

    copies = []
    for j in range(NCHUNK):
        copies.append(
            pltpu.async_copy(ones_v, shist.at[idx_v.at[j]], sem, add=True))
    for cp in copies:
        cp.wait()

    plsc.subcore_barrier()

    @pl.when(s == 0)
    def _():
        pltpu.sync_copy(shist, out_hbm.at[c])


@functools.cache
def _make_hist():
    return pl.kernel(
        _hist_kernel,
        out_type=jax.ShapeDtypeStruct((NC, HTOT), jnp.float32),
        mesh=plsc.VectorSubcoreMesh(core_axis_name="c", subcore_axis_name="s",
                                    num_cores=NC, num_subcores=NS),
        scratch_types=[
            pltpu.VMEM((EPW,), jnp.float32),
            pltpu.VMEM((EPW,), jnp.float32),
            pltpu.VMEM((NCHUNK, 128), jnp.int32),
            pltpu.VMEM((128,), jnp.float32),
            pltpu.VMEM((HTOT // NS,), jnp.float32),
            pltpu.VMEM_SHARED((HTOT,), jnp.float32),
            pltpu.SemaphoreType.DMA,
        ],
    )


def _auc_kernel(hist_ref, out_ref):
    h = hist_ref[0]
    for i in range(1, NC):
        h = h + hist_ref[i]
    fp = h[: HB // 128, :]
    tp = h[HB // 128 :, :]
    r = lax.broadcasted_iota(jnp.int32, (HB // 128, 128), 0)
    col = lax.broadcasted_iota(jnp.int32, (HB // 128, 128), 1)
    live = (r * 128 + col) < NBINS
    fp = jnp.where(live, fp, 0.0)
    tp = jnp.where(live, tp, 0.0)

    ii = lax.broadcasted_iota(jnp.int32, (128, 128), 0)
    jj = lax.broadcasted_iota(jnp.int32, (128, 128), 1)
    u_strict = (ii < jj).astype(jnp.float32)
    pre_in_row = lax.dot(tp, u_strict, precision=lax.Precision.HIGHEST)

    nrow = HB // 128
    rs = jnp.sum(tp, axis=1, keepdims=True)
    aa = lax.broadcasted_iota(jnp.int32, (nrow, nrow), 0)
    bb = lax.broadcasted_iota(jnp.int32, (nrow, nrow), 1)
    l_strict = (bb < aa).astype(jnp.float32)
    row_pre = lax.dot(l_strict, rs, precision=lax.Precision.HIGHEST)

    p_tot = jnp.sum(tp, keepdims=True).reshape(1, 1)
    f_tot = jnp.sum(fp, keepdims=True).reshape(1, 1)
    t_suf = p_tot - (row_pre + pre_in_row)
    num = jnp.sum(fp * (t_suf - 0.5 * tp), keepdims=True).reshape(1, 1)
    out_ref[...] = num / (p_tot * f_tot)


def kernel(preds, targets):
    n = preds.shape[0]
    preds_p = jnp.pad(preds.reshape(-1), (0, NPAD - n))
    targs_p = jnp.pad(targets.reshape(-1), (0, NPAD - n), constant_values=-1.0)
    hist = _make_hist()(preds_p, targs_p)
    hist3 = hist.reshape(NC, 2 * HB // 128, 128)
    auc = pl.pallas_call(
        _auc_kernel,
        out_shape=jax.ShapeDtypeStruct((1, 1), jnp.float32),
    )(hist3)
    return auc[0, 0]


# speedup vs baseline: 1.1715x; 1.1715x over previous
import functools

import jax
import jax.numpy as jnp
from jax import lax
from jax.experimental import pallas as pl
from jax.experimental.pallas import tpu as pltpu
from jax.experimental.pallas import tpu_sc as plsc

NBINS = 10001
HB = 10240
HTOT = 2 * HB
DEAD = HB - 1
NC = 2
NS = 16
NW = NC * NS
EPW = 3200
NPAD = NW * EPW
NCHUNK = EPW // 128
VPC = 128 // 16


def _hist_kernel(preds_hbm, targs_hbm, out_hbm, pred_v, targ_v, idx_v,
                 ones_v, zero_v, shist, sem, insem):
    c = lax.axis_index("c")
    s = lax.axis_index("s")
    wid = s * NC + c
    base = wid * EPW

    in_cp = [
        pltpu.async_copy(preds_hbm.at[pl.ds(base, EPW)], pred_v, insem),
        pltpu.async_copy(targs_hbm.at[pl.ds(base, EPW)], targ_v, insem),
    ]

    ones16 = jnp.full((16,), 1.0, dtype=jnp.float32)
    zero16 = jnp.zeros((16,), dtype=jnp.float32)
    for k in range(128 // 16):
        ones_v[pl.ds(k * 16, 16)] = ones16

    def _zinit(i, _):
        zero_v[pl.ds(i * 16, 16)] = zero16
        return 0
    lax.fori_loop(0, (HTOT // NS) // 16, _zinit, 0)

    stripe = HTOT // NS
    pltpu.sync_copy(zero_v, shist.at[pl.ds(s * stripe, stripe)])

    plsc.subcore_barrier()
    for cp in in_cp:
        cp.wait()

    copies = []
    for j in range(NCHUNK):
        for k in range(VPC):
            off = j * 128 + k * 16
            p = pred_v[pl.ds(off, 16)]
            t = targ_v[pl.ds(off, 16)]
            sg = 1.0 / (1.0 + jnp.exp(-p))
            b = (10000.0 * sg).astype(jnp.int32)
            idx = jnp.where(t >= 0.5, b + HB, b)
            idx = jnp.where(t < -0.5, DEAD, idx)
            idx_v[j, pl.ds(k * 16, 16)] = idx
        copies.append(
            pltpu.async_copy(ones_v, shist.at[idx_v.at[j]], sem, add=True))
    for cp in copies:
        cp.wait()

    plsc.subcore_barrier()

    @pl.when(s == 0)
    def _():
        pltpu.sync_copy(shist, out_hbm.at[c])


@functools.cache
def _make_hist():
    return pl.kernel(
        _hist_kernel,
        out_type=jax.ShapeDtypeStruct((NC, HTOT), jnp.float32),
        mesh=plsc.VectorSubcoreMesh(core_axis_name="c", subcore_axis_name="s",
                                    num_cores=NC, num_subcores=NS),
        scratch_types=[
            pltpu.VMEM((EPW,), jnp.float32),
            pltpu.VMEM((EPW,), jnp.float32),
            pltpu.VMEM((NCHUNK, 128), jnp.int32),
            pltpu.VMEM((128,), jnp.float32),
            pltpu.VMEM((HTOT // NS,), jnp.float32),
            pltpu.VMEM_SHARED((HTOT,), jnp.float32),
            pltpu.SemaphoreType.DMA,
            pltpu.SemaphoreType.DMA,
        ],
    )


def _auc_kernel(hist_ref, out_ref):
    h = hist_ref[0]
    for i in range(1, NC):
        h = h + hist_ref[i]
    fp = h[: HB // 128, :]
    tp = h[HB // 128 :, :]
    r = lax.broadcasted_iota(jnp.int32, (HB // 128, 128), 0)
    col = lax.broadcasted_iota(jnp.int32, (HB // 128, 128), 1)
    live = (r * 128 + col) < NBINS
    fp = jnp.where(live, fp, 0.0)
    tp = jnp.where(live, tp, 0.0)

    ii = lax.broadcasted_iota(jnp.int32, (128, 128), 0)
    jj = lax.broadcasted_iota(jnp.int32, (128, 128), 1)
    u_strict = (ii < jj).astype(jnp.float32)
    pre_in_row = lax.dot(tp, u_strict, precision=lax.Precision.HIGHEST)

    nrow = HB // 128
    rs = jnp.sum(tp, axis=1, keepdims=True)
    aa = lax.broadcasted_iota(jnp.int32, (nrow, nrow), 0)
    bb = lax.broadcasted_iota(jnp.int32, (nrow, nrow), 1)
    l_strict = (bb < aa).astype(jnp.float32)
    row_pre = lax.dot(l_strict, rs, precision=lax.Precision.HIGHEST)

    p_tot = jnp.sum(tp, keepdims=True).reshape(1, 1)
    f_tot = jnp.sum(fp, keepdims=True).reshape(1, 1)
    t_suf = p_tot - (row_pre + pre_in_row)
    num = jnp.sum(fp * (t_suf - 0.5 * tp), keepdims=True).reshape(1, 1)
    out_ref[...] = num / (p_tot * f_tot)


def kernel(preds, targets):
    n = preds.shape[0]
    preds_p = jnp.pad(preds.reshape(-1), (0, NPAD - n))
    targs_p = jnp.pad(targets.reshape(-1), (0, NPAD - n), constant_values=-1.0)
    hist = _make_hist()(preds_p, targs_p)
    hist3 = hist.reshape(NC, 2 * HB // 128, 128)
    auc = pl.pallas_call(
        _auc_kernel,
        out_shape=jax.ShapeDtypeStruct((1, 1), jnp.float32),
    )(hist3)
    return auc[0, 0]


# speedup vs baseline: 1.3178x; 1.1249x over previous
import functools

import jax
import jax.numpy as jnp
from jax import lax
from jax.experimental import pallas as pl
from jax.experimental.pallas import tpu as pltpu
from jax.experimental.pallas import tpu_sc as plsc

N = 100000
NBINS = 10001
HB = 10240
HTOT = 2 * HB
DEAD = HB - 1
NC = 2
NS = 16
NW = NC * NS
EPW = N // NW
WIN = 3200
NCHUNK = WIN // 128
VPC = 128 // 16


def _hist_kernel(preds_hbm, targs_hbm, out_hbm, pred_v, targ_v, idx_v,
                 ones_v, zero_v, shist, sem, insem):
    c = lax.axis_index("c")
    s = lax.axis_index("s")
    wid = s * NC + c
    lo = wid * EPW
    hi = lo + EPW
    base = jnp.minimum((lo // 8) * 8, N - WIN)

    in_cp = [
        pltpu.async_copy(preds_hbm.at[pl.ds(base, WIN)], pred_v, insem),
        pltpu.async_copy(targs_hbm.at[pl.ds(base, WIN)], targ_v, insem),
    ]

    ones16 = jnp.full((16,), 1.0, dtype=jnp.float32)
    zero16 = jnp.zeros((16,), dtype=jnp.float32)
    for k in range(128 // 16):
        ones_v[pl.ds(k * 16, 16)] = ones16

    def _zinit(i, _):
        zero_v[pl.ds(i * 16, 16)] = zero16
        return 0
    lax.fori_loop(0, (HTOT // NS) // 16, _zinit, 0)

    stripe = HTOT // NS
    pltpu.sync_copy(zero_v, shist.at[pl.ds(s * stripe, stripe)])

    plsc.subcore_barrier()
    for cp in in_cp:
        cp.wait()

    lane = lax.iota(jnp.int32, 16)

    def _chunk(j, _):
        for k in range(VPC):
            off = j * 128 + k * 16
            g = base + off + lane
            p = pred_v[pl.ds(off, 16)]
            t = targ_v[pl.ds(off, 16)]
            sg = 1.0 / (1.0 + jnp.exp(-p))
            b = (10000.0 * sg).astype(jnp.int32)
            idx = jnp.where(t >= 0.5, b + HB, b)
            idx = jnp.where((g >= lo) & (g < hi), idx, DEAD)
            idx_v[j, pl.ds(k * 16, 16)] = idx
        pltpu.async_copy(ones_v, shist.at[idx_v.at[j]], sem, add=True)
        return 0
    lax.fori_loop(0, NCHUNK, _chunk, 0)

    pltpu.make_async_copy(preds_hbm.at[pl.ds(0, WIN)], pred_v, sem).wait()

    plsc.subcore_barrier()

    @pl.when(s == 0)
    def _():
        pltpu.sync_copy(shist, out_hbm.at[c])


@functools.cache
def _make_hist():
    return pl.kernel(
        _hist_kernel,
        out_type=jax.ShapeDtypeStruct((NC, HTOT), jnp.float32),
        mesh=plsc.VectorSubcoreMesh(core_axis_name="c", subcore_axis_name="s",
                                    num_cores=NC, num_subcores=NS),
        scratch_types=[
            pltpu.VMEM((WIN,), jnp.float32),
            pltpu.VMEM((WIN,), jnp.float32),
            pltpu.VMEM((NCHUNK, 128), jnp.int32),
            pltpu.VMEM((128,), jnp.float32),
            pltpu.VMEM((HTOT // NS,), jnp.float32),
            pltpu.VMEM_SHARED((HTOT,), jnp.float32),
            pltpu.SemaphoreType.DMA,
            pltpu.SemaphoreType.DMA,
        ],
    )


def _auc_kernel(hist_ref, out_ref):
    h2 = hist_ref[0]
    for i in range(1, NC):
        h2 = h2 + hist_ref[i]
    h = h2.reshape(2 * HB // 128, 128)
    fp = h[: HB // 128, :]
    tp = h[HB // 128 :, :]
    r = lax.broadcasted_iota(jnp.int32, (HB // 128, 128), 0)
    col = lax.broadcasted_iota(jnp.int32, (HB // 128, 128), 1)
    live = (r * 128 + col) < NBINS
    fp = jnp.where(live, fp, 0.0)
    tp = jnp.where(live, tp, 0.0)

    ii = lax.broadcasted_iota(jnp.int32, (128, 128), 0)
    jj = lax.broadcasted_iota(jnp.int32, (128, 128), 1)
    u_strict = (ii < jj).astype(jnp.float32)
    pre_in_row = lax.dot(tp, u_strict, precision=lax.Precision.HIGHEST)

    nrow = HB // 128
    rs = jnp.sum(tp, axis=1, keepdims=True)
    aa = lax.broadcasted_iota(jnp.int32, (nrow, nrow), 0)
    bb = lax.broadcasted_iota(jnp.int32, (nrow, nrow), 1)
    l_strict = (bb < aa).astype(jnp.float32)
    row_pre = lax.dot(l_strict, rs, precision=lax.Precision.HIGHEST)

    p_tot = jnp.sum(tp, keepdims=True).reshape(1, 1)
    f_tot = jnp.sum(fp, keepdims=True).reshape(1, 1)
    t_suf = p_tot - (row_pre + pre_in_row)
    num = jnp.sum(fp * (t_suf - 0.5 * tp), keepdims=True).reshape(1, 1)
    out_ref[...] = num / (p_tot * f_tot)


def kernel(preds, targets):
    hist = _make_hist()(preds.reshape(-1), targets.reshape(-1))
    auc = pl.pallas_call(
        _auc_kernel,
        out_shape=jax.ShapeDtypeStruct((1, 1), jnp.float32),
    )(hist)
    return auc[0, 0]
